# baseline (speedup 1.0000x reference)
import functools

import jax
import jax.numpy as jnp
from jax import lax
from jax.experimental import pallas as pl
from jax.experimental.pallas import tpu as pltpu
from jax.experimental.pallas import tpu_sc as plsc

NUM_CODES = 8192
CODE_DIM = 256
BETA = 0.25
EPS = 1e-6

TM = 384


def _norm_rows(x, eps=EPS):
    n = jnp.sqrt(jnp.sum(x * x, axis=1, keepdims=True))
    return x / jnp.maximum(n, eps)


def _fused_body(h_ref, cb_ref, idx_ref, cbn_ref, q_ref, ct_ref, loss_ref,
                csq_ref, hn_all, idx_all):
    i = pl.program_id(0)
    ni = pl.num_programs(0) - 1

    @pl.when(i == 0)
    def _():
        cbn = _norm_rows(cb_ref[...])
        cbn_ref[...] = cbn
        csq_ref[0, :] = jnp.sum(cbn * cbn, axis=1)

    @pl.when(i < ni)
    def _():
        hn = _norm_rows(h_ref[...])
        hn_all[pl.ds(i * TM, TM), :] = hn
        h_sq = jnp.sum(hn * hn, axis=1, keepdims=True)
        NCH, CW = 4, NUM_CODES // 4
        bv = bi = None
        for k in range(NCH):
            cbn_k = cbn_ref[pl.ds(k * CW, CW), :]
            dot_k = lax.dot_general(hn, cbn_k, (((1,), (1,)), ((), ())),
                                    preferred_element_type=jnp.float32)
            dist_k = (h_sq + csq_ref[0, pl.ds(k * CW, CW)][None, :]) - 2.0 * dot_k
            amin_k = jnp.argmin(dist_k, axis=1).astype(jnp.int32) + k * CW
            min_k = jnp.min(dist_k, axis=1)
            if k == 0:
                bv, bi = min_k, amin_k
            else:
                take = min_k < bv
                bv = jnp.where(take, min_k, bv)
                bi = jnp.where(take, amin_k, bi)
        amin = bi
        idx_all[pl.ds(i * TM, TM)] = amin
        idx_ref[0, 0, :] = amin

    off = jnp.maximum(i - 1, 0) * TM
    idx_v = idx_all[pl.ds(off, TM)]
    cols = lax.broadcasted_iota(jnp.int32, (TM, NUM_CODES), 1)
    q = (idx_v[:, None] == cols).astype(jnp.float32)
    q_ref[...] = q
    ct = lax.dot_general(q, cbn_ref[...], (((1,), (0,)), ((), ())),
                         preferred_element_type=jnp.float32)
    ct_ref[...] = ct
    d = hn_all[pl.ds(off, TM), :] - ct
    per_tok = jnp.sum(d * d, axis=1)
    loss_ref[0, 0, :] = jnp.sum(per_tok.reshape(-1, 128), axis=0)


def _make_sc_gather(T, D):
    info = plsc.get_sparse_core_info()
    NC, NS = info.num_cores, info.num_subcores
    NW = NC * NS
    b_per_w = T // NW
    chunk = b_per_w
    while chunk > 128:
        chunk //= 2
    n_chunks = b_per_w // chunk
    mesh = plsc.VectorSubcoreMesh(core_axis_name="c", subcore_axis_name="s")

    @functools.partial(
        pl.kernel, mesh=mesh,
        out_type=jax.ShapeDtypeStruct((T, D), jnp.float32),
        scratch_types=[
            pltpu.VMEM((b_per_w,), jnp.int32),
            pltpu.VMEM((b_per_w, D), jnp.float32),
            pltpu.SemaphoreType.DMA,
            pltpu.SemaphoreType.DMA,
        ],
    )
    def sc_gather(table_hbm, idx_hbm, out_hbm, idx_v, rows_v, sem, sem2):
        wid = lax.axis_index("s") * NC + lax.axis_index("c")
        base = wid * b_per_w
        pltpu.sync_copy(idx_hbm.at[pl.ds(base, b_per_w)], idx_v)
        gathers = [
            pltpu.async_copy(
                table_hbm.at[idx_v.at[pl.ds(k * chunk, chunk)]],
                rows_v.at[pl.ds(k * chunk, chunk), :], sem)
            for k in range(n_chunks)
        ]
        outs = []
        for k in range(n_chunks):
            gathers[k].wait()
            outs.append(pltpu.async_copy(
                rows_v.at[pl.ds(k * chunk, chunk), :],
                out_hbm.at[pl.ds(base + k * chunk, chunk)], sem2))
        for c in outs:
            c.wait()

    return sc_gather


def kernel(h, codebook):
    B0, B1, D = h.shape
    T = B0 * B1
    h_flat = h.reshape(T, D)
    ni = T // TM

    idx3, cbn, q2, c_tilde, loss_p = pl.pallas_call(
        _fused_body,
        grid=(ni + 1,),
        in_specs=[
            pl.BlockSpec((TM, D), lambda i: (jnp.minimum(i, ni - 1), 0)),
            pl.BlockSpec((NUM_CODES, D), lambda i: (0, 0)),
        ],
        out_specs=[
            pl.BlockSpec((1, 1, TM), lambda i: (jnp.minimum(i, ni - 1), 0, 0)),
            pl.BlockSpec((NUM_CODES, D), lambda i: (0, 0)),
            pl.BlockSpec((TM, NUM_CODES), lambda i: (jnp.maximum(i - 1, 0), 0)),
            pl.BlockSpec((TM, D), lambda i: (jnp.maximum(i - 1, 0), 0)),
            pl.BlockSpec((1, 1, 128), lambda i: (jnp.maximum(i - 1, 0), 0, 0)),
        ],
        out_shape=[
            jax.ShapeDtypeStruct((ni, 1, TM), jnp.int32),
            jax.ShapeDtypeStruct((NUM_CODES, D), jnp.float32),
            jax.ShapeDtypeStruct((T, NUM_CODES), jnp.float32),
            jax.ShapeDtypeStruct((T, D), jnp.float32),
            jax.ShapeDtypeStruct((ni, 1, 128), jnp.float32),
        ],
        scratch_shapes=[
            pltpu.VMEM((1, NUM_CODES), jnp.float32),
            pltpu.VMEM((T, D), jnp.float32),
            pltpu.VMEM((T,), jnp.int32),
        ],
    )(h_flat, codebook)

    indices_flat = idx3.reshape(T)
    ch_exact = _make_sc_gather(T, D)(cbn, indices_flat)

    q = q2.reshape(B0, B1, NUM_CODES)
    ct3 = c_tilde.reshape(B0, B1, D)
    ch3 = ch_exact.reshape(B0, B1, D)
    m = jnp.sum(loss_p[:, 0, :]) / jnp.float32(T * D)
    loss = m + BETA * m
    return (q, ct3, ch3, ch3, loss, indices_flat)


# speedup vs baseline: 1.0756x; 1.0756x over previous
import functools

import jax
import jax.numpy as jnp
from jax import lax
from jax.experimental import pallas as pl
from jax.experimental.pallas import tpu as pltpu
from jax.experimental.pallas import tpu_sc as plsc

NUM_CODES = 8192
CODE_DIM = 256
BETA = 0.25
EPS = 1e-6

TM = 384


def _norm_rows(x, eps=EPS):
    n = jnp.sqrt(jnp.sum(x * x, axis=1, keepdims=True))
    return x / jnp.maximum(n, eps)


def _fused_body(h_ref, cb_ref, idx_ref, cbn_ref, q_ref, ct_ref, loss_ref,
                csq_ref, hn_all, idx_all):
    i = pl.program_id(0)
    ni = pl.num_programs(0) - 1

    @pl.when(i == 0)
    def _():
        cbn = _norm_rows(cb_ref[...])
        cbn_ref[...] = cbn
        csq_ref[0, :] = jnp.sum(cbn * cbn, axis=1)

    @pl.when(i < ni)
    def _():
        hn = _norm_rows(h_ref[...])
        hn_all[pl.ds(i * TM, TM), :] = hn
        h_sq = jnp.sum(hn * hn, axis=1, keepdims=True)
        dot = lax.dot_general(hn, cbn_ref[...], (((1,), (1,)), ((), ())),
                              preferred_element_type=jnp.float32)
        dist = (h_sq + csq_ref[0, :][None, :]) - 2.0 * dot
        amin = jnp.argmin(dist, axis=1).astype(jnp.int32)
        idx_all[pl.ds(i * TM, TM)] = amin
        idx_ref[0, 0, :] = amin

    off = jnp.maximum(i - 1, 0) * TM
    idx_v = idx_all[pl.ds(off, TM)]
    cols = lax.broadcasted_iota(jnp.int32, (TM, NUM_CODES), 1)
    q = (idx_v[:, None] == cols).astype(jnp.float32)
    q_ref[...] = q
    ct = lax.dot_general(q, cbn_ref[...], (((1,), (0,)), ((), ())),
                         preferred_element_type=jnp.float32)
    ct_ref[...] = ct
    d = hn_all[pl.ds(off, TM), :] - ct
    per_tok = jnp.sum(d * d, axis=1)
    loss_ref[0, 0, :] = jnp.sum(per_tok.reshape(-1, 128), axis=0)


def _make_sc_gather(T, D):
    info = plsc.get_sparse_core_info()
    NC, NS = info.num_cores, info.num_subcores
    NW = NC * NS
    b_per_w = T // NW
    chunk = b_per_w
    while chunk > 128:
        chunk //= 2
    n_chunks = b_per_w // chunk
    mesh = plsc.VectorSubcoreMesh(core_axis_name="c", subcore_axis_name="s")

    @functools.partial(
        pl.kernel, mesh=mesh,
        out_type=jax.ShapeDtypeStruct((T, D), jnp.float32),
        scratch_types=[
            pltpu.VMEM((b_per_w,), jnp.int32),
            pltpu.VMEM((b_per_w, D), jnp.float32),
            pltpu.SemaphoreType.DMA,
            pltpu.SemaphoreType.DMA,
        ],
    )
    def sc_gather(table_hbm, idx_hbm, out_hbm, idx_v, rows_v, sem, sem2):
        wid = lax.axis_index("s") * NC + lax.axis_index("c")
        base = wid * b_per_w
        pltpu.sync_copy(idx_hbm.at[pl.ds(base, b_per_w)], idx_v)
        gathers = [
            pltpu.async_copy(
                table_hbm.at[idx_v.at[pl.ds(k * chunk, chunk)]],
                rows_v.at[pl.ds(k * chunk, chunk), :], sem)
            for k in range(n_chunks)
        ]
        outs = []
        for k in range(n_chunks):
            gathers[k].wait()
            outs.append(pltpu.async_copy(
                rows_v.at[pl.ds(k * chunk, chunk), :],
                out_hbm.at[pl.ds(base + k * chunk, chunk)], sem2))
        for c in outs:
            c.wait()

    return sc_gather


def kernel(h, codebook):
    B0, B1, D = h.shape
    T = B0 * B1
    h_flat = h.reshape(T, D)
    ni = T // TM

    idx3, cbn, q2, c_tilde, loss_p = pl.pallas_call(
        _fused_body,
        grid=(ni + 1,),
        in_specs=[
            pl.BlockSpec((TM, D), lambda i: (jnp.minimum(i, ni - 1), 0)),
            pl.BlockSpec((NUM_CODES, D), lambda i: (0, 0)),
        ],
        out_specs=[
            pl.BlockSpec((1, 1, TM), lambda i: (jnp.minimum(i, ni - 1), 0, 0)),
            pl.BlockSpec((NUM_CODES, D), lambda i: (0, 0)),
            pl.BlockSpec((TM, NUM_CODES), lambda i: (jnp.maximum(i - 1, 0), 0)),
            pl.BlockSpec((TM, D), lambda i: (jnp.maximum(i - 1, 0), 0)),
            pl.BlockSpec((1, 1, 128), lambda i: (jnp.maximum(i - 1, 0), 0, 0)),
        ],
        out_shape=[
            jax.ShapeDtypeStruct((ni, 1, TM), jnp.int32),
            jax.ShapeDtypeStruct((NUM_CODES, D), jnp.float32),
            jax.ShapeDtypeStruct((T, NUM_CODES), jnp.float32),
            jax.ShapeDtypeStruct((T, D), jnp.float32),
            jax.ShapeDtypeStruct((ni, 1, 128), jnp.float32),
        ],
        scratch_shapes=[
            pltpu.VMEM((1, NUM_CODES), jnp.float32),
            pltpu.VMEM((T, D), jnp.float32),
            pltpu.VMEM((T,), jnp.int32),
        ],
    )(h_flat, codebook)

    indices_flat = idx3.reshape(T)
    ch_exact = _make_sc_gather(T, D)(cbn, indices_flat)

    q = q2.reshape(B0, B1, NUM_CODES)
    ct3 = c_tilde.reshape(B0, B1, D)
    ch3 = ch_exact.reshape(B0, B1, D)
    m = jnp.sum(loss_p[:, 0, :]) / jnp.float32(T * D)
    loss = m + BETA * m
    return (q, ct3, ch3, ch3, loss, indices_flat)
